# Initial kernel scaffold; baseline (speedup 1.0000x reference)
import functools

import jax
import jax.numpy as jnp
import numpy as np
from jax import lax
from jax.experimental import pallas as pl
from jax.experimental.pallas import tpu as pltpu
from jax.experimental.pallas import tpu_sc as plsc

_N = 10000
_E = 320000
_D = 128
_H = 64

_NC = 2
_NS = 16
_NW = _NC * _NS
_EPW = _E // _NW
_CB = 80
_NPT = _N // _NS
_NR = _N // 16

_NEG = np.asarray(jax.random.randint(jax.random.key(42), (2, 5 * _N), 0, _N))
_NEG_CNT = float(np.sum(_NEG[0] < _NEG[1]))

_TE = 371200
_ETW = _TE // _NW
_RID = np.arange(_NR, dtype=np.int32).reshape(5, 125)

_mesh = plsc.VectorSubcoreMesh(core_axis_name="c", subcore_axis_name="s")


@functools.partial(
    pl.kernel,
    out_type=jax.ShapeDtypeStruct((_NC, _NR, 16), jnp.float32),
    mesh=_mesh,
    scratch_types=[
        pltpu.VMEM((_EPW,), jnp.int32),
        pltpu.VMEM((_NR, 16), jnp.float32),
        pltpu.VMEM((5, 125), jnp.int32),
        pltpu.VMEM_SHARED((_NR, 16), jnp.float32),
    ],
)
def _deg_sc(row_hbm, z_hbm, rid_hbm, out_hbm, idx_v, deg_v, rid_v, deg_sh):
    cid = lax.axis_index("c")
    sid = lax.axis_index("s")
    base = (cid * _NS + sid) * _EPW
    pltpu.sync_copy(row_hbm.at[pl.ds(base, _EPW)], idx_v)
    pltpu.sync_copy(rid_hbm, rid_v)
    pltpu.sync_copy(z_hbm, deg_v)

    @pl.when(sid < 5)
    def _zero_shared():
        pltpu.sync_copy(z_hbm.at[pl.ds(sid * 125, 125)],
                        deg_sh.at[pl.ds(sid * 125, 125)])

    plsc.subcore_barrier()

    ones = jnp.ones((16,), jnp.float32)

    def body(k, carry):
        g = idx_v[pl.ds(k * 16, 16)]
        r = lax.shift_right_logical(g, 4)
        c = jnp.bitwise_and(g, 15)
        plsc.addupdate_scatter(deg_v, [r, c], ones)
        return carry

    lax.fori_loop(0, _EPW // 16, body, 0)

    for t in range(5):
        pltpu.sync_copy(deg_v.at[pl.ds(t * 125, 125)],
                        deg_sh.at[rid_v.at[t]], add=True)
    plsc.subcore_barrier()

    @pl.when(sid == 0)
    def _copy_out():
        pltpu.sync_copy(deg_sh, out_hbm.at[cid])


@functools.partial(
    pl.kernel,
    out_type=jax.ShapeDtypeStruct((_NC, _N, _H), jnp.float32),
    mesh=_mesh,
    scratch_types=[
        pltpu.VMEM((_CB,), jnp.int32),
        pltpu.VMEM((1, _CB), jnp.int32),
        pltpu.VMEM((_CB, _H), jnp.float32),
        pltpu.VMEM_SHARED((_N, _H), jnp.float32),
        pltpu.SemaphoreType.DMA,
    ],
)
def _agg_sc(y_hbm, row_hbm, col_hbm, z_hbm, out_hbm, colv, rowv, rows, acc, sem):
    cid = lax.axis_index("c")
    sid = lax.axis_index("s")
    base = (cid * _NS + sid) * _EPW
    pltpu.sync_copy(z_hbm.at[pl.ds(sid * _NPT, _NPT)],
                    acc.at[pl.ds(sid * _NPT, _NPT)])
    plsc.subcore_barrier()

    def body(k, carry):
        off = base + k * _CB
        pltpu.sync_copy(col_hbm.at[pl.ds(off, _CB)], colv)
        pltpu.sync_copy(row_hbm.at[pl.ds(off, _CB)], rowv.at[0])
        pltpu.async_copy(y_hbm.at[colv], rows, sem).wait()
        pltpu.sync_copy(rows, acc.at[rowv.at[0]], add=True)
        return carry

    lax.fori_loop(0, _EPW // _CB, body, 0)
    plsc.subcore_barrier()
    pltpu.sync_copy(acc.at[pl.ds(sid * _NPT, _NPT)],
                    out_hbm.at[cid, pl.ds(sid * _NPT, _NPT)])


@functools.partial(
    pl.kernel,
    out_type=jax.ShapeDtypeStruct((_NC, _NS, 4, 16), jnp.float32),
    mesh=_mesh,
    scratch_types=[
        pltpu.VMEM((_CB,), jnp.int32),
        pltpu.VMEM((_CB,), jnp.int32),
        pltpu.VMEM((_CB, _H), jnp.float32),
        pltpu.VMEM((_CB, _H), jnp.float32),
        pltpu.VMEM((4, 16), jnp.float32),
        pltpu.SemaphoreType.DMA,
        pltpu.SemaphoreType.DMA,
    ],
)
def _rec_sc(reps_hbm, u_hbm, v_hbm, out_hbm, uv, vv, xb, yb, stag, semx, semy):
    cid = lax.axis_index("c")
    sid = lax.axis_index("s")
    base = (cid * _NS + sid) * _ETW
    zeros = jnp.zeros((16,), jnp.float32)
    for r in range(4):
        stag[r] = zeros

    def body(k, carry):
        off = base + k * _CB
        pltpu.sync_copy(u_hbm.at[pl.ds(off, _CB)], uv)
        pltpu.sync_copy(v_hbm.at[pl.ds(off, _CB)], vv)
        cx = pltpu.async_copy(reps_hbm.at[uv], xb, semx)
        cy = pltpu.async_copy(reps_hbm.at[vv], yb, semy)
        cx.wait()
        cy.wait()
        for g in range(5):
            u16 = uv[pl.ds(g * 16, 16)]
            v16 = vv[pl.ds(g * 16, 16)]
            mask = u16 < v16
            rows16 = lax.iota(jnp.int32, 16) + g * 16
            s = jnp.zeros((16,), jnp.float32)
            for d in range(_H):
                cols = jnp.full((16,), d, jnp.int32)
                s = s + plsc.load_gather(xb, [rows16, cols]) * \
                    plsc.load_gather(yb, [rows16, cols])
            is_pos = (off + g * 16) < _E
            term = jnp.where(mask,
                             jnp.where(is_pos, (s - 1.0) ** 2, s * s),
                             0.0)
            stag[0] = stag[0] + jnp.where(is_pos, term, 0.0)
            stag[1] = stag[1] + jnp.where(is_pos, 0.0, term)
            stag[2] = stag[2] + jnp.where(
                jnp.logical_and(mask, is_pos), 1.0, 0.0)
        return carry

    lax.fori_loop(0, _ETW // _CB, body, 0)
    pltpu.sync_copy(stag, out_hbm.at[cid, sid])


_RB = 400


def _dinv_of(deg_blk):
    d = deg_blk[:, 0:1] + deg_blk[:, 1:2] + 1.0
    return lax.rsqrt(d)


def _tc1_body(x_ref, w_ref, deg_ref, y_ref):
    dinv = _dinv_of(deg_ref[...])
    y_ref[...] = dinv * jnp.dot(x_ref[...], w_ref[...],
                                preferred_element_type=jnp.float32)


def _tc2_body(acc_a, acc_b, y1_ref, deg_ref, b1_ref, w2_ref, y2_ref):
    dinv = _dinv_of(deg_ref[...])
    h = dinv * (acc_a[...] + acc_b[...] + y1_ref[...]) + b1_ref[...]
    h = jnp.maximum(h, 0.0)
    y2_ref[...] = dinv * jnp.dot(h, w2_ref[...],
                                 preferred_element_type=jnp.float32)


def _tc3_body(acc_a, acc_b, y2_ref, deg_ref, b2_ref, reps_ref):
    dinv = _dinv_of(deg_ref[...])
    reps_ref[...] = dinv * (acc_a[...] + acc_b[...] + y2_ref[...]) + b2_ref[...]


def _tc4_body(p_ref, out_ref):
    pos = jnp.sum(p_ref[0])
    neg = jnp.sum(p_ref[1])
    cnt = jnp.sum(p_ref[2])
    loss = (neg + pos) * float(_N) / (cnt + _NEG_CNT)
    out_ref[...] = jnp.reshape(loss, (1, 1))


_acc_spec_a = pl.BlockSpec((None, _RB, _H), lambda i: (0, i, 0))
_acc_spec_b = pl.BlockSpec((None, _RB, _H), lambda i: (1, i, 0))
_nh_spec = pl.BlockSpec((_RB, _H), lambda i: (i, 0))
_deg_spec = pl.BlockSpec((_RB, 2), lambda i: (i, 0))
_bias_spec = pl.BlockSpec((1, _H), lambda i: (0, 0))

_tc1 = pl.pallas_call(
    _tc1_body,
    grid=(_N // _RB,),
    in_specs=[
        pl.BlockSpec((_RB, _D), lambda i: (i, 0)),
        pl.BlockSpec((_D, _H), lambda i: (0, 0)),
        _deg_spec,
    ],
    out_specs=_nh_spec,
    out_shape=jax.ShapeDtypeStruct((_N, _H), jnp.float32),
)

_tc2 = pl.pallas_call(
    _tc2_body,
    grid=(_N // _RB,),
    in_specs=[
        _acc_spec_a, _acc_spec_b, _nh_spec, _deg_spec, _bias_spec,
        pl.BlockSpec((_H, _H), lambda i: (0, 0)),
    ],
    out_specs=_nh_spec,
    out_shape=jax.ShapeDtypeStruct((_N, _H), jnp.float32),
)

_tc3 = pl.pallas_call(
    _tc3_body,
    grid=(_N // _RB,),
    in_specs=[_acc_spec_a, _acc_spec_b, _nh_spec, _deg_spec, _bias_spec],
    out_specs=_nh_spec,
    out_shape=jax.ShapeDtypeStruct((_N, _H), jnp.float32),
)

_tc4 = pl.pallas_call(
    _tc4_body,
    grid=(1,),
    in_specs=[pl.BlockSpec((4, _NW * 16), lambda i: (0, 0))],
    out_specs=pl.BlockSpec((1, 1), lambda i: (0, 0)),
    out_shape=jax.ShapeDtypeStruct((1, 1), jnp.float32),
)


def kernel(edge_index, features, W1, b1, W2, b2):
    row = edge_index[0]
    col = edge_index[1]
    zeros_nh = jnp.zeros((_N, _H), jnp.float32)
    zeros_deg = jnp.zeros((_NR, 16), jnp.float32)
    rid = jnp.asarray(_RID)

    deg2 = _deg_sc(row, zeros_deg, rid)
    deg_t = jnp.transpose(deg2.reshape(_NC, _N))

    y1 = _tc1(features, W1, deg_t)
    acc1 = _agg_sc(y1, row, col, zeros_nh)
    y2 = _tc2(acc1, acc1, y1, deg_t, b1.reshape(1, _H), W2)
    acc2 = _agg_sc(y2, row, col, zeros_nh)
    reps = _tc3(acc2, acc2, y2, deg_t, b2.reshape(1, _H))

    pad = jnp.zeros((_TE - _E - 5 * _N,), jnp.int32)
    u_all = jnp.concatenate([row, jnp.asarray(_NEG[0], jnp.int32), pad])
    v_all = jnp.concatenate([col, jnp.asarray(_NEG[1], jnp.int32), pad])
    partials = _rec_sc(reps, u_all, v_all)
    p = jnp.transpose(partials.reshape(_NW, 4, 16), (1, 0, 2)).reshape(4, _NW * 16)
    loss = _tc4(p)
    return (reps, jnp.reshape(loss, ()))


# speedup vs baseline: 6.7822x; 6.7822x over previous
import functools

import jax
import jax.numpy as jnp
import numpy as np
from jax import lax
from jax.experimental import pallas as pl
from jax.experimental.pallas import tpu as pltpu
from jax.experimental.pallas import tpu_sc as plsc

_N = 10000
_E = 320000
_D = 128
_H = 64

_NC = 2
_NS = 16
_NW = _NC * _NS
_EPW = _E // _NW
_CB = 80
_NPT = 640
_NP = _NS * _NPT
_NR = _N // 16
_NRP = 640

_TE = 371200
_ETW = _TE // _NW

_mesh = plsc.VectorSubcoreMesh(core_axis_name="c", subcore_axis_name="s",
                               num_cores=_NC, num_subcores=_NS)
_sc_params = pltpu.CompilerParams(use_tc_tiling_on_sc=False)


@functools.partial(
    pl.kernel,
    out_type=jax.ShapeDtypeStruct((_NC, _NP, 16), jnp.float32),
    mesh=_mesh,
    compiler_params=_sc_params,
    scratch_types=[
        pltpu.VMEM((1, _CB), jnp.int32),
        pltpu.VMEM((_CB, 16), jnp.float32),
        pltpu.VMEM_SHARED((_NP, 16), jnp.float32),
    ],
)
def _deg_sc(row_hbm, ones_hbm, z_hbm, out_hbm, rowv, ones_v, acc):
    cid = lax.axis_index("c")
    sid = lax.axis_index("s")
    base = (cid * _NS + sid) * _EPW
    pltpu.sync_copy(ones_hbm, ones_v)
    pltpu.sync_copy(z_hbm.at[pl.ds(sid * _NPT, _NPT)],
                    acc.at[pl.ds(sid * _NPT, _NPT)])
    plsc.subcore_barrier()

    def body(k, carry):
        off = base + k * _CB
        pltpu.sync_copy(row_hbm.at[pl.ds(off, _CB)], rowv.at[0])
        pltpu.sync_copy(ones_v, acc.at[rowv.at[0]], add=True)
        return carry

    lax.fori_loop(0, _EPW // _CB, body, 0)
    plsc.subcore_barrier()
    pltpu.sync_copy(acc.at[pl.ds(sid * _NPT, _NPT)],
                    out_hbm.at[cid, pl.ds(sid * _NPT, _NPT)])


@functools.partial(
    pl.kernel,
    out_type=jax.ShapeDtypeStruct((_NC, _NP, _H), jnp.float32),
    mesh=_mesh,
    compiler_params=_sc_params,
    scratch_types=[
        pltpu.VMEM((_CB,), jnp.int32),
        pltpu.VMEM((1, _CB), jnp.int32),
        pltpu.VMEM((_CB, _H), jnp.float32),
        pltpu.VMEM_SHARED((_NP, _H), jnp.float32),
        pltpu.SemaphoreType.DMA,
    ],
)
def _agg_sc(y_hbm, row_hbm, col_hbm, z_hbm, out_hbm, colv, rowv, rows, acc, sem):
    cid = lax.axis_index("c")
    sid = lax.axis_index("s")
    base = (cid * _NS + sid) * _EPW
    pltpu.sync_copy(z_hbm.at[pl.ds(sid * _NPT, _NPT)],
                    acc.at[pl.ds(sid * _NPT, _NPT)])
    plsc.subcore_barrier()

    def body(k, carry):
        off = base + k * _CB
        pltpu.sync_copy(col_hbm.at[pl.ds(off, _CB)], colv)
        pltpu.sync_copy(row_hbm.at[pl.ds(off, _CB)], rowv.at[0])
        pltpu.async_copy(y_hbm.at[colv], rows, sem).wait()
        pltpu.sync_copy(rows, acc.at[rowv.at[0]], add=True)
        return carry

    lax.fori_loop(0, _EPW // _CB, body, 0)
    plsc.subcore_barrier()
    pltpu.sync_copy(acc.at[pl.ds(sid * _NPT, _NPT)],
                    out_hbm.at[cid, pl.ds(sid * _NPT, _NPT)])


@functools.partial(
    pl.kernel,
    out_type=jax.ShapeDtypeStruct((_TE, 16), jnp.float32),
    mesh=_mesh,
    compiler_params=_sc_params,
    scratch_types=[
        pltpu.VMEM((_CB,), jnp.int32),
        pltpu.VMEM((_CB,), jnp.int32),
        pltpu.VMEM((_CB, _H), jnp.float32),
        pltpu.VMEM((_CB, _H), jnp.float32),
        pltpu.VMEM((_CB, 16), jnp.float32),
        pltpu.SemaphoreType.DMA,
        pltpu.SemaphoreType.DMA,
    ],
)
def _rec_sc(reps_hbm, u_hbm, v_hbm, out_hbm, uv, vv, xb, yb, vout, semx, semy):
    cid = lax.axis_index("c")
    sid = lax.axis_index("s")
    base = (cid * _NS + sid) * _ETW

    def body(k, carry):
        off = base + k * _CB
        pltpu.sync_copy(u_hbm.at[pl.ds(off, _CB)], uv)
        pltpu.sync_copy(v_hbm.at[pl.ds(off, _CB)], vv)
        cx = pltpu.async_copy(reps_hbm.at[uv], xb, semx)
        cy = pltpu.async_copy(reps_hbm.at[vv], yb, semy)
        cx.wait()
        cy.wait()
        for j in range(_CB):
            p = xb[j, pl.ds(0, 16)] * yb[j, pl.ds(0, 16)]
            for q in range(1, 4):
                p = p + xb[j, pl.ds(q * 16, 16)] * yb[j, pl.ds(q * 16, 16)]
            vout[j] = p
        pltpu.sync_copy(vout, out_hbm.at[pl.ds(off, _CB)])
        return carry

    lax.fori_loop(0, _ETW // _CB, body, 0)


_RB = 400


def _dinv_of(deg_blk):
    d = deg_blk[:, 0:1] + deg_blk[:, 1:2] + 1.0
    return lax.rsqrt(d)


def _tc1_body(x_ref, w_ref, deg_ref, y_ref):
    dinv = _dinv_of(deg_ref[...])
    y_ref[...] = dinv * jnp.dot(x_ref[...], w_ref[...],
                                preferred_element_type=jnp.float32)


def _tc2_body(acc_a, acc_b, y1_ref, deg_ref, b1_ref, w2_ref, y2_ref):
    dinv = _dinv_of(deg_ref[...])
    h = dinv * (acc_a[...] + acc_b[...] + y1_ref[...]) + b1_ref[...]
    h = jnp.maximum(h, 0.0)
    y2_ref[...] = dinv * jnp.dot(h, w2_ref[...],
                                 preferred_element_type=jnp.float32)


def _tc3_body(acc_a, acc_b, y2_ref, deg_ref, b2_ref, reps_ref):
    dinv = _dinv_of(deg_ref[...])
    reps_ref[...] = dinv * (acc_a[...] + acc_b[...] + y2_ref[...]) + b2_ref[...]


_RBE = 1600
_NBE = _TE // _RBE


def _tcr_body(vs_ref, u_ref, v_ref, out_ref):
    i = pl.program_id(0)
    s = jnp.sum(vs_ref[...], axis=1, keepdims=True)
    mask = u_ref[...] < v_ref[...]
    eidx = i * _RBE + lax.broadcasted_iota(jnp.int32, (_RBE, 1), 0)
    is_pos = eidx < _E
    sm1 = s - 1.0
    term = jnp.where(mask, jnp.where(is_pos, sm1 * sm1, s * s), 0.0)
    cntv = jnp.where(mask, 1.0, 0.0)
    vals = (jnp.sum(jnp.where(is_pos, term, 0.0)) *
            (lax.broadcasted_iota(jnp.int32, (1, 4), 1) == 0) +
            jnp.sum(jnp.where(is_pos, 0.0, term)) *
            (lax.broadcasted_iota(jnp.int32, (1, 4), 1) == 1) +
            jnp.sum(jnp.where(is_pos, cntv, 0.0)) *
            (lax.broadcasted_iota(jnp.int32, (1, 4), 1) == 2) +
            jnp.sum(jnp.where(is_pos, 0.0, cntv)) *
            (lax.broadcasted_iota(jnp.int32, (1, 4), 1) == 3))
    out_ref[...] = vals


def _tc4_body(p_ref, out_ref):
    p = p_ref[...]
    pos = jnp.sum(p[:, 0])
    neg = jnp.sum(p[:, 1])
    cnt = jnp.sum(p[:, 2]) + jnp.sum(p[:, 3])
    loss = (neg + pos) * float(_N) / cnt
    out_ref[...] = jnp.reshape(loss, (1, 1))


_acc_spec_a = pl.BlockSpec((None, _RB, _H), lambda i: (0, i, 0))
_acc_spec_b = pl.BlockSpec((None, _RB, _H), lambda i: (1, i, 0))
_nh_spec = pl.BlockSpec((_RB, _H), lambda i: (i, 0))
_deg_spec = pl.BlockSpec((_RB, 2), lambda i: (i, 0))
_bias_spec = pl.BlockSpec((1, _H), lambda i: (0, 0))

_tc1 = pl.pallas_call(
    _tc1_body,
    grid=(_N // _RB,),
    in_specs=[
        pl.BlockSpec((_RB, _D), lambda i: (i, 0)),
        pl.BlockSpec((_D, _H), lambda i: (0, 0)),
        _deg_spec,
    ],
    out_specs=_nh_spec,
    out_shape=jax.ShapeDtypeStruct((_N, _H), jnp.float32),
)

_tc2 = pl.pallas_call(
    _tc2_body,
    grid=(_N // _RB,),
    in_specs=[
        _acc_spec_a, _acc_spec_b, _nh_spec, _deg_spec, _bias_spec,
        pl.BlockSpec((_H, _H), lambda i: (0, 0)),
    ],
    out_specs=_nh_spec,
    out_shape=jax.ShapeDtypeStruct((_N, _H), jnp.float32),
)

_tc3 = pl.pallas_call(
    _tc3_body,
    grid=(_N // _RB,),
    in_specs=[_acc_spec_a, _acc_spec_b, _nh_spec, _deg_spec, _bias_spec],
    out_specs=_nh_spec,
    out_shape=jax.ShapeDtypeStruct((_N, _H), jnp.float32),
)

_tcr = pl.pallas_call(
    _tcr_body,
    grid=(_NBE,),
    in_specs=[
        pl.BlockSpec((_RBE, 16), lambda i: (i, 0)),
        pl.BlockSpec((_RBE, 1), lambda i: (i, 0)),
        pl.BlockSpec((_RBE, 1), lambda i: (i, 0)),
    ],
    out_specs=pl.BlockSpec((None, 1, 4), lambda i: (i, 0, 0)),
    out_shape=jax.ShapeDtypeStruct((_NBE, 1, 4), jnp.float32),
)

_tc4 = pl.pallas_call(
    _tc4_body,
    grid=(1,),
    in_specs=[pl.BlockSpec((_NBE, 4), lambda i: (0, 0))],
    out_specs=pl.BlockSpec((1, 1), lambda i: (0, 0)),
    out_shape=jax.ShapeDtypeStruct((1, 1), jnp.float32),
)


def kernel(edge_index, features, W1, b1, W2, b2):
    row = edge_index[0]
    col = edge_index[1]
    zeros_nh = jnp.zeros((_NP, _H), jnp.float32)
    zeros_deg = jnp.zeros((_NP, 16), jnp.float32)
    ones_cb = jnp.ones((_CB, 16), jnp.float32)

    deg2 = _deg_sc(row, ones_cb, zeros_deg)
    deg_t = jnp.transpose(deg2[:, :_N, 0])

    y1 = _tc1(features, W1, deg_t)
    acc1 = _agg_sc(y1, row, col, zeros_nh)
    y2 = _tc2(acc1, acc1, y1, deg_t, b1.reshape(1, _H), W2)
    acc2 = _agg_sc(y2, row, col, zeros_nh)
    reps = _tc3(acc2, acc2, y2, deg_t, b2.reshape(1, _H))

    neg = jax.random.randint(jax.random.key(42), (2, 5 * _N), 0, _N)
    pad = jnp.zeros((_TE - _E - 5 * _N,), jnp.int32)
    u_all = jnp.concatenate([row, neg[0], pad])
    v_all = jnp.concatenate([col, neg[1], pad])
    vsum = _rec_sc(reps, u_all, v_all)
    partials = _tcr(vsum,
                    u_all.astype(jnp.float32).reshape(_TE, 1),
                    v_all.astype(jnp.float32).reshape(_TE, 1))
    loss = _tc4(partials.reshape(_NBE, 4))
    return (reps, jnp.reshape(loss, ()))


# speedup vs baseline: 10.3642x; 1.5281x over previous
import functools

import jax
import jax.numpy as jnp
import numpy as np
from jax import lax
from jax.experimental import pallas as pl
from jax.experimental.pallas import tpu as pltpu
from jax.experimental.pallas import tpu_sc as plsc

_N = 10000
_E = 320000
_D = 128
_H = 64

_NC = 2
_NS = 16
_NW = _NC * _NS
_EPW = _E // _NW
_CB = 80
_DCB = 1000
_NPT = 640
_NP = _NS * _NPT

_TE = 371200
_ETW = _TE // _NW

_mesh = plsc.VectorSubcoreMesh(core_axis_name="c", subcore_axis_name="s",
                               num_cores=_NC, num_subcores=_NS)
_sc_params = pltpu.CompilerParams(use_tc_tiling_on_sc=False)

_NEG = np.asarray(jax.random.randint(jax.random.key(42), (2, 5 * _N), 0, _N),
                  np.int32)
_PAD = np.zeros((_TE - _E - 5 * _N,), np.int32)
_UV_TAIL = np.concatenate([_NEG, np.broadcast_to(_PAD, (2, _PAD.shape[0]))],
                          axis=1)


@functools.partial(
    pl.kernel,
    out_type=jax.ShapeDtypeStruct((_NC, _NP, 16), jnp.float32),
    mesh=_mesh,
    compiler_params=_sc_params,
    scratch_types=[
        pltpu.VMEM((_EPW,), jnp.int32),
        pltpu.VMEM((_DCB, 16), jnp.float32),
        pltpu.VMEM_SHARED((_NP, 16), jnp.float32),
    ],
)
def _deg_sc(row_hbm, ones_hbm, z_hbm, out_hbm, rowv, ones_v, acc):
    cid = lax.axis_index("c")
    sid = lax.axis_index("s")
    base = (cid * _NS + sid) * _EPW
    pltpu.sync_copy(row_hbm.at[pl.ds(base, _EPW)], rowv)
    pltpu.sync_copy(ones_hbm, ones_v)
    pltpu.sync_copy(z_hbm.at[pl.ds(sid * _NPT, _NPT)],
                    acc.at[pl.ds(sid * _NPT, _NPT)])
    plsc.subcore_barrier()

    def body(k, carry):
        pltpu.sync_copy(ones_v, acc.at[rowv.at[pl.ds(k * _DCB, _DCB)]],
                        add=True)
        return carry

    lax.fori_loop(0, _EPW // _DCB, body, 0)
    plsc.subcore_barrier()
    pltpu.sync_copy(acc.at[pl.ds(sid * _NPT, _NPT)],
                    out_hbm.at[cid, pl.ds(sid * _NPT, _NPT)])


_NAC = _EPW // _CB
_NAP = (_NAC - 1) // 2


@functools.partial(
    pl.kernel,
    out_type=jax.ShapeDtypeStruct((_NC, _NP, _H), jnp.float32),
    mesh=_mesh,
    compiler_params=_sc_params,
    scratch_types=[
        pltpu.VMEM((_EPW,), jnp.int32),
        pltpu.VMEM((_EPW,), jnp.int32),
        pltpu.VMEM((2, _CB, _H), jnp.float32),
        pltpu.VMEM_SHARED((_NP, _H), jnp.float32),
        pltpu.SemaphoreType.DMA,
        pltpu.SemaphoreType.DMA,
    ],
)
def _agg_sc(y_hbm, row_hbm, col_hbm, z_hbm, out_hbm,
            colv, rowv, rows, acc, sem0, sem1):
    cid = lax.axis_index("c")
    sid = lax.axis_index("s")
    base = (cid * _NS + sid) * _EPW
    pltpu.sync_copy(col_hbm.at[pl.ds(base, _EPW)], colv)
    pltpu.sync_copy(row_hbm.at[pl.ds(base, _EPW)], rowv)
    pltpu.sync_copy(z_hbm.at[pl.ds(sid * _NPT, _NPT)],
                    acc.at[pl.ds(sid * _NPT, _NPT)])
    plsc.subcore_barrier()

    def gidx(c):
        return colv.at[pl.ds(c * _CB, _CB)]

    def sidx(c):
        return rowv.at[pl.ds(c * _CB, _CB)]

    def start(c, buf, sem):
        pltpu.make_async_copy(y_hbm.at[gidx(c)], rows.at[buf], sem).start()

    def finish(c, buf, sem):
        pltpu.make_async_copy(y_hbm.at[gidx(c)], rows.at[buf], sem).wait()
        pltpu.sync_copy(rows.at[buf], acc.at[sidx(c)], add=True)

    start(0, 0, sem0)

    def body(i, carry):
        c0 = 2 * i
        start(c0 + 1, 1, sem1)
        finish(c0, 0, sem0)
        start(c0 + 2, 0, sem0)
        finish(c0 + 1, 1, sem1)
        return carry

    lax.fori_loop(0, _NAP, body, 0)
    finish(_NAC - 1, 0, sem0)
    plsc.subcore_barrier()
    pltpu.sync_copy(acc.at[pl.ds(sid * _NPT, _NPT)],
                    out_hbm.at[cid, pl.ds(sid * _NPT, _NPT)])


_NRC = _ETW // _CB
_NRQ = (_NRC - 1) // 2


@functools.partial(
    pl.kernel,
    out_type=jax.ShapeDtypeStruct((_TE, 16), jnp.float32),
    mesh=_mesh,
    compiler_params=_sc_params,
    scratch_types=[
        pltpu.VMEM((_ETW,), jnp.int32),
        pltpu.VMEM((_ETW,), jnp.int32),
        pltpu.VMEM((2, _CB, _H), jnp.float32),
        pltpu.VMEM((2, _CB, _H), jnp.float32),
        pltpu.VMEM((2, _CB, 16), jnp.float32),
        pltpu.SemaphoreType.DMA,
        pltpu.SemaphoreType.DMA,
        pltpu.SemaphoreType.DMA,
        pltpu.SemaphoreType.DMA,
    ],
)
def _rec_sc(reps_hbm, u_hbm, v_hbm, out_hbm,
            uv, vv, xb, yb, vout, semx0, semy0, semx1, semy1):
    cid = lax.axis_index("c")
    sid = lax.axis_index("s")
    base = (cid * _NS + sid) * _ETW
    pltpu.sync_copy(u_hbm.at[pl.ds(base, _ETW)], uv)
    pltpu.sync_copy(v_hbm.at[pl.ds(base, _ETW)], vv)

    def start(c, buf, semx, semy):
        ui = uv.at[pl.ds(c * _CB, _CB)]
        vi = vv.at[pl.ds(c * _CB, _CB)]
        pltpu.make_async_copy(reps_hbm.at[ui], xb.at[buf], semx).start()
        pltpu.make_async_copy(reps_hbm.at[vi], yb.at[buf], semy).start()

    def finish(c, buf, semx, semy):
        ui = uv.at[pl.ds(c * _CB, _CB)]
        vi = vv.at[pl.ds(c * _CB, _CB)]
        pltpu.make_async_copy(reps_hbm.at[ui], xb.at[buf], semx).wait()
        pltpu.make_async_copy(reps_hbm.at[vi], yb.at[buf], semy).wait()
        for j in range(_CB):
            p = xb[buf, j, pl.ds(0, 16)] * yb[buf, j, pl.ds(0, 16)]
            for q in range(1, 4):
                p = p + (xb[buf, j, pl.ds(q * 16, 16)] *
                         yb[buf, j, pl.ds(q * 16, 16)])
            vout[buf, j] = p
        pltpu.sync_copy(vout.at[buf], out_hbm.at[pl.ds(base + c * _CB, _CB)])

    start(0, 0, semx0, semy0)

    def body(i, carry):
        c0 = 2 * i
        start(c0 + 1, 1, semx1, semy1)
        finish(c0, 0, semx0, semy0)
        start(c0 + 2, 0, semx0, semy0)
        finish(c0 + 1, 1, semx1, semy1)
        return carry

    lax.fori_loop(0, _NRQ, body, 0)
    finish(_NRC - 1, 0, semx0, semy0)


_RB = 400


def _dinv_of(deg_blk):
    d = deg_blk[:, 0:1] + deg_blk[:, 1:2] + 1.0
    return lax.rsqrt(d)


def _tc1_body(x_ref, w_ref, deg_ref, y_ref):
    dinv = _dinv_of(deg_ref[...])
    y_ref[...] = dinv * jnp.dot(x_ref[...], w_ref[...],
                                preferred_element_type=jnp.float32)


def _tc2_body(acc_a, acc_b, y1_ref, deg_ref, b1_ref, w2_ref, y2_ref):
    dinv = _dinv_of(deg_ref[...])
    h = dinv * (acc_a[...] + acc_b[...] + y1_ref[...]) + b1_ref[...]
    h = jnp.maximum(h, 0.0)
    y2_ref[...] = dinv * jnp.dot(h, w2_ref[...],
                                 preferred_element_type=jnp.float32)


def _tc3_body(acc_a, acc_b, y2_ref, deg_ref, b2_ref, reps_ref):
    dinv = _dinv_of(deg_ref[...])
    reps_ref[...] = dinv * (acc_a[...] + acc_b[...] + y2_ref[...]) + b2_ref[...]


_RBE = 1600
_NBE = _TE // _RBE


def _tcr_body(vs_ref, u_ref, v_ref, out_ref):
    i = pl.program_id(0)
    s = jnp.sum(vs_ref[...], axis=1, keepdims=True)
    mask = u_ref[...] < v_ref[...]
    eidx = i * _RBE + lax.broadcasted_iota(jnp.int32, (_RBE, 1), 0)
    is_pos = eidx < _E
    sm1 = s - 1.0
    term = jnp.where(mask, jnp.where(is_pos, sm1 * sm1, s * s), 0.0)
    cntv = jnp.where(mask, 1.0, 0.0)
    vals = (jnp.sum(jnp.where(is_pos, term, 0.0)) *
            (lax.broadcasted_iota(jnp.int32, (1, 4), 1) == 0) +
            jnp.sum(jnp.where(is_pos, 0.0, term)) *
            (lax.broadcasted_iota(jnp.int32, (1, 4), 1) == 1) +
            jnp.sum(jnp.where(is_pos, cntv, 0.0)) *
            (lax.broadcasted_iota(jnp.int32, (1, 4), 1) == 2) +
            jnp.sum(jnp.where(is_pos, 0.0, cntv)) *
            (lax.broadcasted_iota(jnp.int32, (1, 4), 1) == 3))
    out_ref[...] = vals


def _tc4_body(p_ref, out_ref):
    p = p_ref[...]
    pos = jnp.sum(p[:, 0])
    neg = jnp.sum(p[:, 1])
    cnt = jnp.sum(p[:, 2]) + jnp.sum(p[:, 3])
    loss = (neg + pos) * float(_N) / cnt
    out_ref[...] = jnp.reshape(loss, (1, 1))


_acc_spec_a = pl.BlockSpec((None, _RB, _H), lambda i: (0, i, 0))
_acc_spec_b = pl.BlockSpec((None, _RB, _H), lambda i: (1, i, 0))
_nh_spec = pl.BlockSpec((_RB, _H), lambda i: (i, 0))
_deg_spec = pl.BlockSpec((_RB, 2), lambda i: (i, 0))
_bias_spec = pl.BlockSpec((1, _H), lambda i: (0, 0))

_tc1 = pl.pallas_call(
    _tc1_body,
    grid=(_N // _RB,),
    in_specs=[
        pl.BlockSpec((_RB, _D), lambda i: (i, 0)),
        pl.BlockSpec((_D, _H), lambda i: (0, 0)),
        _deg_spec,
    ],
    out_specs=_nh_spec,
    out_shape=jax.ShapeDtypeStruct((_N, _H), jnp.float32),
)

_tc2 = pl.pallas_call(
    _tc2_body,
    grid=(_N // _RB,),
    in_specs=[
        _acc_spec_a, _acc_spec_b, _nh_spec, _deg_spec, _bias_spec,
        pl.BlockSpec((_H, _H), lambda i: (0, 0)),
    ],
    out_specs=_nh_spec,
    out_shape=jax.ShapeDtypeStruct((_N, _H), jnp.float32),
)

_tc3 = pl.pallas_call(
    _tc3_body,
    grid=(_N // _RB,),
    in_specs=[_acc_spec_a, _acc_spec_b, _nh_spec, _deg_spec, _bias_spec],
    out_specs=_nh_spec,
    out_shape=jax.ShapeDtypeStruct((_N, _H), jnp.float32),
)

_tcr = pl.pallas_call(
    _tcr_body,
    grid=(_NBE,),
    in_specs=[
        pl.BlockSpec((_RBE, 16), lambda i: (i, 0)),
        pl.BlockSpec((_RBE, 1), lambda i: (i, 0)),
        pl.BlockSpec((_RBE, 1), lambda i: (i, 0)),
    ],
    out_specs=pl.BlockSpec((None, 1, 4), lambda i: (i, 0, 0)),
    out_shape=jax.ShapeDtypeStruct((_NBE, 1, 4), jnp.float32),
)

_tc4 = pl.pallas_call(
    _tc4_body,
    grid=(1,),
    in_specs=[pl.BlockSpec((_NBE, 4), lambda i: (0, 0))],
    out_specs=pl.BlockSpec((1, 1), lambda i: (0, 0)),
    out_shape=jax.ShapeDtypeStruct((1, 1), jnp.float32),
)


def kernel(edge_index, features, W1, b1, W2, b2):
    row = edge_index[0]
    col = edge_index[1]
    zeros_nh = jnp.zeros((_NP, _H), jnp.float32)
    zeros_deg = jnp.zeros((_NP, 16), jnp.float32)
    ones_dcb = jnp.ones((_DCB, 16), jnp.float32)

    deg2 = _deg_sc(row, ones_dcb, zeros_deg)
    deg_t = jnp.transpose(deg2[:, :_N, 0])

    y1 = _tc1(features, W1, deg_t)
    acc1 = _agg_sc(y1, row, col, zeros_nh)
    y2 = _tc2(acc1, acc1, y1, deg_t, b1.reshape(1, _H), W2)
    acc2 = _agg_sc(y2, row, col, zeros_nh)
    reps = _tc3(acc2, acc2, y2, deg_t, b2.reshape(1, _H))

    u_all = jnp.concatenate([row, jnp.asarray(_UV_TAIL[0])])
    v_all = jnp.concatenate([col, jnp.asarray(_UV_TAIL[1])])
    vsum = _rec_sc(reps, u_all, v_all)
    partials = _tcr(vsum,
                    u_all.astype(jnp.float32).reshape(_TE, 1),
                    v_all.astype(jnp.float32).reshape(_TE, 1))
    loss = _tc4(partials.reshape(_NBE, 4))
    return (reps, jnp.reshape(loss, ()))
